# baseline (speedup 1.0000x reference)
import functools

import jax
import jax.numpy as jnp
from jax import lax
from jax.experimental import pallas as pl
from jax.experimental.pallas import tpu as pltpu
from jax.experimental.pallas import tpu_sc as plsc

C = 1000
D = 256
N = 16384
MOM = 0.95
TAU = 0.1
EPS = 1e-8

NC, NS = 2, 16
NW = NC * NS
K = 128
CW = 64
NCS = D // CW
NRG = NW // NCS
GROWS = N // NRG
NCH = GROWS // K

_GATHER_DNUMS = lax.GatherDimensionNumbers(
    offset_dims=(), collapsed_slice_dims=(0,), start_index_map=(0,))


def _splat(vec, k16):
    idx = jnp.full((16, 1), k16, jnp.int32)
    return lax.gather(
        vec, idx, _GATHER_DNUMS, (1,),
        mode=lax.GatherScatterMode.PROMISE_IN_BOUNDS)


def _sc_scatter(feats, weights, labels):
    mesh = plsc.VectorSubcoreMesh(core_axis_name="c", subcore_axis_name="s")

    @functools.partial(
        pl.kernel,
        out_type=(
            jax.ShapeDtypeStruct((NRG, NCS, 2, C * 32), jnp.float32),
            jax.ShapeDtypeStruct((NRG, NCS, C * 16), jnp.float32),
        ),
        mesh=mesh,
        scratch_types=[
            pltpu.VMEM((C * 32,), jnp.float32),
            pltpu.VMEM((C * 32,), jnp.float32),
            pltpu.VMEM((C * 16,), jnp.float32),
            pltpu.VMEM((K, 2 * CW), jnp.float32),
            pltpu.VMEM((K, 2 * CW), jnp.float32),
            pltpu.VMEM((GROWS,), jnp.int32),
            pltpu.VMEM((GROWS,), jnp.float32),
            pltpu.SemaphoreType.DMA,
        ],
        compiler_params=pltpu.CompilerParams(needs_layout_passes=False),
    )
    def k(feats_hbm, w_hbm, lbl_hbm, vsum_hbm, wsum_hbm,
          acc_a, acc_b, wacc_v, buf_a, buf_b, lbl_v, w_v, sem):
        c = lax.axis_index("c")
        s = lax.axis_index("s")
        wid = c * NS + s
        cs = wid % NCS
        rg = wid // NCS
        sub = (cs % 2) * CW
        lane_iota = lax.iota(jnp.int32, 16)
        row0 = rg * GROWS

        def feats_copy(t, buf):
            return pltpu.make_async_copy(
                feats_hbm.at[pl.ds(row0 + t * K, K),
                             pl.ds((cs // 2) * 128, 128)],
                buf, sem)

        feats_copy(0, buf_a).start()
        feats_copy(1, buf_b).start()
        pltpu.sync_copy(lbl_hbm.at[pl.ds(row0, GROWS)], lbl_v)
        pltpu.sync_copy(w_hbm.at[pl.ds(row0, GROWS)], w_v)

        def zero_acc(i, _):
            for u in range(8):
                acc_a[pl.ds((i * 8 + u) * 16, 16)] = jnp.zeros(
                    (16,), jnp.float32)
                acc_b[pl.ds((i * 8 + u) * 16, 16)] = jnp.zeros(
                    (16,), jnp.float32)
            for u in range(4):
                wacc_v[pl.ds((i * 4 + u) * 16, 16)] = jnp.zeros(
                    (16,), jnp.float32)
            return 0
        lax.fori_loop(0, C * 32 // (16 * 8), zero_acc, 0)

        def accumulate(t, buf):
            for g in range(K // 16):
                lblg = lbl_v[pl.ds(t * K + g * 16, 16)]
                wg = w_v[pl.ds(t * K + g * 16, 16)]
                for k16 in range(16):
                    lspl = _splat(lblg, k16)
                    wspl = _splat(wg, k16)
                    base32 = lspl * 32
                    addr0 = base32 + lane_iota
                    addr1 = addr0 + 16
                    r = g * 16 + k16
                    v0 = buf[r, pl.ds(sub, 16)] * wspl
                    v2 = buf[r, pl.ds(sub + 32, 16)] * wspl
                    v1 = buf[r, pl.ds(sub + 16, 16)] * wspl
                    v3 = buf[r, pl.ds(sub + 48, 16)] * wspl
                    plsc.addupdate_scatter(acc_a, [addr0], v0)
                    plsc.addupdate_scatter(acc_b, [addr0], v2)
                    plsc.addupdate_scatter(acc_a, [addr1], v1)
                    plsc.addupdate_scatter(acc_b, [addr1], v3)

        def vsum_pair(u, _):
            t = 2 * u
            feats_copy(t, buf_a).wait()
            accumulate(t, buf_a)

            @pl.when(t + 2 < NCH)
            def _():
                feats_copy(t + 2, buf_a).start()

            feats_copy(t + 1, buf_b).wait()
            accumulate(t + 1, buf_b)

            @pl.when(t + 3 < NCH)
            def _():
                feats_copy(t + 3, buf_b).start()
            return 0
        lax.fori_loop(0, NCH // 2, vsum_pair, 0)

        def wsum_grp(g, _):
            off = cs * (GROWS // NCS) + g * 16
            lblg = lbl_v[pl.ds(off, 16)]
            wg = w_v[pl.ds(off, 16)]
            for k16 in range(16):
                lspl = _splat(lblg, k16)
                wspl = _splat(wg, k16)
                waddr = lspl * 16 + lane_iota
                plsc.addupdate_scatter(wacc_v, [waddr], wspl)
            return 0
        lax.fori_loop(0, GROWS // NCS // 16, wsum_grp, 0)

        pltpu.sync_copy(acc_a, vsum_hbm.at[rg, cs, 0])
        pltpu.sync_copy(acc_b, vsum_hbm.at[rg, cs, 1])
        pltpu.sync_copy(wacc_v, wsum_hbm.at[rg, cs])

    return k(feats, weights, labels)


def _tc_reduce(vsum, wsum):
    def body(vs_ref, ws_ref, vo_ref, wo_ref):
        vo_ref[...] = jnp.sum(vs_ref[...], axis=0)
        wo_ref[...] = jnp.sum(ws_ref[...], axis=(0, 1))

    return pl.pallas_call(
        body,
        out_shape=[
            jax.ShapeDtypeStruct((NCS, 2, C * 32), jnp.float32),
            jax.ShapeDtypeStruct((C * 16,), jnp.float32),
        ],
    )(vsum, wsum)


def _tc_proto(vsum, wsum, proto):
    def body(vs_ref, ws_ref, p_ref, out_ref):
        vs = jnp.concatenate(
            [vs_ref[i, j] for i in range(NCS) for j in range(2)],
            axis=1)
        ws = ws_ref[...][:, 0:1]
        vec = vs / jnp.clip(ws, EPS, None)
        old = p_ref[...]
        newp = jnp.where(ws > 0.0, MOM * old + (1.0 - MOM) * vec, old)
        nrm = jnp.sqrt(jnp.sum(newp * newp, axis=1, keepdims=True))
        out_ref[...] = newp / jnp.clip(nrm, EPS, None)

    return pl.pallas_call(
        body,
        out_shape=jax.ShapeDtypeStruct((C, D), jnp.float32),
    )(vsum, wsum, proto)


def _tc_logits(feats, p):
    BN = 4096

    def body(f_ref, p_ref, out_ref):
        f = f_ref[...]
        nrm = jnp.sqrt(jnp.sum(f * f, axis=1, keepdims=True))
        fn = f / jnp.clip(nrm, EPS, None)
        acc = lax.dot_general(
            p_ref[...], fn, (((1,), (1,)), ((), ())),
            preferred_element_type=jnp.float32,
            precision=lax.Precision.DEFAULT,
        )
        out_ref[...] = acc / TAU

    out_t = pl.pallas_call(
        body,
        grid=(N // BN,),
        in_specs=[
            pl.BlockSpec((BN, D), lambda i: (i, 0)),
            pl.BlockSpec((C, D), lambda i: (0, 0)),
        ],
        out_specs=pl.BlockSpec((C, BN), lambda i: (0, i)),
        out_shape=jax.ShapeDtypeStruct((C, N), jnp.float32),
    )(feats, p)
    return out_t.T


def kernel(feats, labels, weights, proto):
    labels = labels.astype(jnp.int32)
    vsum, wsum = _sc_scatter(feats, weights, labels)
    vsum_r, wsum_r = _tc_reduce(vsum, wsum)
    p = _tc_proto(vsum_r.reshape(NCS, 2, C, 32), wsum_r.reshape(C, 16), proto)
    return _tc_logits(feats, p)


# speedup vs baseline: 1.2027x; 1.2027x over previous
import functools

import jax
import jax.numpy as jnp
from jax import lax
from jax.experimental import pallas as pl
from jax.experimental.pallas import tpu as pltpu
from jax.experimental.pallas import tpu_sc as plsc

C = 1000
D = 256
N = 16384
MOM = 0.95
TAU = 0.1
EPS = 1e-8

NC, NS = 2, 16
NW = NC * NS
K = 128
CW = 64
NCS = D // CW
NRG = NW // NCS
GROWS = N // NRG
NCH = GROWS // K

_GATHER_DNUMS = lax.GatherDimensionNumbers(
    offset_dims=(), collapsed_slice_dims=(0,), start_index_map=(0,))


def _splat(vec, k16):
    idx = jnp.full((16, 1), k16, jnp.int32)
    return lax.gather(
        vec, idx, _GATHER_DNUMS, (1,),
        mode=lax.GatherScatterMode.PROMISE_IN_BOUNDS)


def _sc_scatter(feats, weights, labels):
    mesh = plsc.VectorSubcoreMesh(core_axis_name="c", subcore_axis_name="s")

    @functools.partial(
        pl.kernel,
        out_type=(
            jax.ShapeDtypeStruct((NRG, NCS, 2, C * 32), jnp.float32),
            jax.ShapeDtypeStruct((NRG, NCS, C * 16), jnp.float32),
        ),
        mesh=mesh,
        scratch_types=[
            pltpu.VMEM((C * 32,), jnp.float32),
            pltpu.VMEM((C * 32,), jnp.float32),
            pltpu.VMEM((C * 16,), jnp.float32),
            pltpu.VMEM((2, K, 2 * CW), jnp.float32),
            pltpu.VMEM((GROWS,), jnp.int32),
            pltpu.VMEM((GROWS,), jnp.float32),
            pltpu.SemaphoreType.DMA,
        ],
        compiler_params=pltpu.CompilerParams(needs_layout_passes=False),
    )
    def k(feats_hbm, w_hbm, lbl_hbm, vsum_hbm, wsum_hbm,
          acc_a, acc_b, wacc_v, buf_v, lbl_v, w_v, sem):
        c = lax.axis_index("c")
        s = lax.axis_index("s")
        wid = c * NS + s
        cs = wid % NCS
        rg = wid // NCS
        sub = (cs % 2) * CW
        lane_iota = lax.iota(jnp.int32, 16)
        row0 = rg * GROWS

        def feats_copy(t, p):
            return pltpu.make_async_copy(
                feats_hbm.at[pl.ds(row0 + t * K, K),
                             pl.ds((cs // 2) * 128, 128)],
                buf_v.at[p], sem)

        feats_copy(0, 0).start()
        pltpu.sync_copy(lbl_hbm.at[pl.ds(row0, GROWS)], lbl_v)
        pltpu.sync_copy(w_hbm.at[pl.ds(row0, GROWS)], w_v)

        def zero_acc(i, _):
            for u in range(8):
                acc_a[pl.ds((i * 8 + u) * 16, 16)] = jnp.zeros(
                    (16,), jnp.float32)
                acc_b[pl.ds((i * 8 + u) * 16, 16)] = jnp.zeros(
                    (16,), jnp.float32)
            for u in range(4):
                wacc_v[pl.ds((i * 4 + u) * 16, 16)] = jnp.zeros(
                    (16,), jnp.float32)
            return 0
        lax.fori_loop(0, C * 32 // (16 * 8), zero_acc, 0)

        def vsum_chunk(t, _):
            p = lax.rem(t, 2)

            @pl.when(t + 1 < NCH)
            def _():
                feats_copy(t + 1, 1 - p).start()

            feats_copy(t, p).wait()
            for g in range(K // 16):
                lblg = lbl_v[pl.ds(t * K + g * 16, 16)]
                wg = w_v[pl.ds(t * K + g * 16, 16)]
                for k16 in range(16):
                    lspl = _splat(lblg, k16)
                    wspl = _splat(wg, k16)
                    base32 = lspl * 32
                    addr0 = base32 + lane_iota
                    addr1 = addr0 + 16
                    r = g * 16 + k16
                    v0 = buf_v[p, r, pl.ds(sub, 16)] * wspl
                    v2 = buf_v[p, r, pl.ds(sub + 32, 16)] * wspl
                    v1 = buf_v[p, r, pl.ds(sub + 16, 16)] * wspl
                    v3 = buf_v[p, r, pl.ds(sub + 48, 16)] * wspl
                    plsc.addupdate_scatter(acc_a, [addr0], v0)
                    plsc.addupdate_scatter(acc_b, [addr0], v2)
                    plsc.addupdate_scatter(acc_a, [addr1], v1)
                    plsc.addupdate_scatter(acc_b, [addr1], v3)
            return 0
        lax.fori_loop(0, NCH, vsum_chunk, 0)

        def wsum_grp(g, _):
            off = cs * (GROWS // NCS) + g * 16
            lblg = lbl_v[pl.ds(off, 16)]
            wg = w_v[pl.ds(off, 16)]
            for k16 in range(16):
                lspl = _splat(lblg, k16)
                wspl = _splat(wg, k16)
                waddr = lspl * 16 + lane_iota
                plsc.addupdate_scatter(wacc_v, [waddr], wspl)
            return 0
        lax.fori_loop(0, GROWS // NCS // 16, wsum_grp, 0)

        pltpu.sync_copy(acc_a, vsum_hbm.at[rg, cs, 0])
        pltpu.sync_copy(acc_b, vsum_hbm.at[rg, cs, 1])
        pltpu.sync_copy(wacc_v, wsum_hbm.at[rg, cs])

    return k(feats, weights, labels)


def _tc_reduce(vsum, wsum):
    def body(vs_ref, ws_ref, vo_ref, wo_ref):
        vo_ref[...] = jnp.sum(vs_ref[...], axis=0)
        wo_ref[...] = jnp.sum(ws_ref[...], axis=(0, 1))

    return pl.pallas_call(
        body,
        out_shape=[
            jax.ShapeDtypeStruct((NCS, 2, C * 32), jnp.float32),
            jax.ShapeDtypeStruct((C * 16,), jnp.float32),
        ],
    )(vsum, wsum)


def _tc_proto(vsum, wsum, proto):
    def body(vs_ref, ws_ref, p_ref, out_ref):
        vs = jnp.concatenate(
            [vs_ref[i, j] for i in range(NCS) for j in range(2)],
            axis=1)
        ws = ws_ref[...][:, 0:1]
        vec = vs / jnp.clip(ws, EPS, None)
        old = p_ref[...]
        newp = jnp.where(ws > 0.0, MOM * old + (1.0 - MOM) * vec, old)
        nrm = jnp.sqrt(jnp.sum(newp * newp, axis=1, keepdims=True))
        out_ref[...] = newp / jnp.clip(nrm, EPS, None)

    return pl.pallas_call(
        body,
        out_shape=jax.ShapeDtypeStruct((C, D), jnp.float32),
    )(vsum, wsum, proto)


def _tc_logits(feats, p):
    BN = 4096

    def body(f_ref, p_ref, out_ref):
        f = f_ref[...]
        nrm = jnp.sqrt(jnp.sum(f * f, axis=1, keepdims=True))
        fn = f / jnp.clip(nrm, EPS, None)
        acc = lax.dot_general(
            p_ref[...], fn, (((1,), (1,)), ((), ())),
            preferred_element_type=jnp.float32,
            precision=lax.Precision.DEFAULT,
        )
        out_ref[...] = acc / TAU

    out_t = pl.pallas_call(
        body,
        grid=(N // BN,),
        in_specs=[
            pl.BlockSpec((BN, D), lambda i: (i, 0)),
            pl.BlockSpec((C, D), lambda i: (0, 0)),
        ],
        out_specs=pl.BlockSpec((C, BN), lambda i: (0, i)),
        out_shape=jax.ShapeDtypeStruct((C, N), jnp.float32),
    )(feats, p)
    return out_t.T


def kernel(feats, labels, weights, proto):
    labels = labels.astype(jnp.int32)
    vsum, wsum = _sc_scatter(feats, weights, labels)
    vsum_r, wsum_r = _tc_reduce(vsum, wsum)
    p = _tc_proto(vsum_r.reshape(NCS, 2, C, 32), wsum_r.reshape(C, 16), proto)
    return _tc_logits(feats, p)


# speedup vs baseline: 1.2397x; 1.0308x over previous
import functools

import jax
import jax.numpy as jnp
from jax import lax
from jax.experimental import pallas as pl
from jax.experimental.pallas import tpu as pltpu
from jax.experimental.pallas import tpu_sc as plsc

C = 1000
D = 256
N = 16384
MOM = 0.95
TAU = 0.1
EPS = 1e-8

NC, NS = 2, 16
NW = NC * NS
K = 128
CW = 64
NCS = D // CW
NRG = NW // NCS
GROWS = N // NRG
NCH = GROWS // K

_GATHER_DNUMS = lax.GatherDimensionNumbers(
    offset_dims=(), collapsed_slice_dims=(0,), start_index_map=(0,))


def _splat(vec, k16):
    idx = jnp.full((16, 1), k16, jnp.int32)
    return lax.gather(
        vec, idx, _GATHER_DNUMS, (1,),
        mode=lax.GatherScatterMode.PROMISE_IN_BOUNDS)


def _sc_scatter(feats, weights, labels):
    mesh = plsc.VectorSubcoreMesh(core_axis_name="c", subcore_axis_name="s")

    @functools.partial(
        pl.kernel,
        out_type=(
            jax.ShapeDtypeStruct((NRG, NCS, 4, C * 16), jnp.float32),
            jax.ShapeDtypeStruct((NRG, NCS, C * 16), jnp.float32),
        ),
        mesh=mesh,
        scratch_types=[
            pltpu.VMEM((C * 16,), jnp.float32),
            pltpu.VMEM((C * 16,), jnp.float32),
            pltpu.VMEM((C * 16,), jnp.float32),
            pltpu.VMEM((C * 16,), jnp.float32),
            pltpu.VMEM((C * 16,), jnp.float32),
            pltpu.VMEM((2, K, 2 * CW), jnp.float32),
            pltpu.VMEM((GROWS,), jnp.int32),
            pltpu.VMEM((GROWS,), jnp.float32),
            pltpu.SemaphoreType.DMA,
        ],
        compiler_params=pltpu.CompilerParams(needs_layout_passes=False),
    )
    def k(feats_hbm, w_hbm, lbl_hbm, vsum_hbm, wsum_hbm,
          acc_a, acc_b, acc_c, acc_d, wacc_v, buf_v, lbl_v, w_v, sem):
        c = lax.axis_index("c")
        s = lax.axis_index("s")
        wid = c * NS + s
        cs = wid % NCS
        rg = wid // NCS
        sub = (cs % 2) * CW
        lane_iota = lax.iota(jnp.int32, 16)
        row0 = rg * GROWS

        def feats_copy(t, p):
            return pltpu.make_async_copy(
                feats_hbm.at[pl.ds(row0 + t * K, K),
                             pl.ds((cs // 2) * 128, 128)],
                buf_v.at[p], sem)

        feats_copy(0, 0).start()
        pltpu.sync_copy(lbl_hbm.at[pl.ds(row0, GROWS)], lbl_v)
        pltpu.sync_copy(w_hbm.at[pl.ds(row0, GROWS)], w_v)

        def zero_acc(i, _):
            for u in range(4):
                off = pl.ds((i * 4 + u) * 16, 16)
                z = jnp.zeros((16,), jnp.float32)
                acc_a[off] = z
                acc_b[off] = z
                acc_c[off] = z
                acc_d[off] = z
                wacc_v[off] = z
            return 0
        lax.fori_loop(0, C * 16 // (16 * 4), zero_acc, 0)

        def vsum_chunk(t, _):
            p = lax.rem(t, 2)

            @pl.when(t + 1 < NCH)
            def _():
                feats_copy(t + 1, 1 - p).start()

            feats_copy(t, p).wait()
            for g in range(K // 16):
                lblg = lbl_v[pl.ds(t * K + g * 16, 16)]
                wg = w_v[pl.ds(t * K + g * 16, 16)]
                for k16 in range(16):
                    lspl = _splat(lblg, k16)
                    wspl = _splat(wg, k16)
                    addr = lspl * 16 + lane_iota
                    r = g * 16 + k16
                    v0 = buf_v[p, r, pl.ds(sub, 16)] * wspl
                    v1 = buf_v[p, r, pl.ds(sub + 16, 16)] * wspl
                    v2 = buf_v[p, r, pl.ds(sub + 32, 16)] * wspl
                    v3 = buf_v[p, r, pl.ds(sub + 48, 16)] * wspl
                    plsc.addupdate_scatter(acc_a, [addr], v0)
                    plsc.addupdate_scatter(acc_b, [addr], v1)
                    plsc.addupdate_scatter(acc_c, [addr], v2)
                    plsc.addupdate_scatter(acc_d, [addr], v3)
            return 0
        lax.fori_loop(0, NCH, vsum_chunk, 0)

        def wsum_grp(g, _):
            off = cs * (GROWS // NCS) + g * 16
            lblg = lbl_v[pl.ds(off, 16)]
            wg = w_v[pl.ds(off, 16)]
            for k16 in range(16):
                lspl = _splat(lblg, k16)
                wspl = _splat(wg, k16)
                waddr = lspl * 16 + lane_iota
                plsc.addupdate_scatter(wacc_v, [waddr], wspl)
            return 0
        lax.fori_loop(0, GROWS // NCS // 16, wsum_grp, 0)

        pltpu.sync_copy(acc_a, vsum_hbm.at[rg, cs, 0])
        pltpu.sync_copy(acc_b, vsum_hbm.at[rg, cs, 1])
        pltpu.sync_copy(acc_c, vsum_hbm.at[rg, cs, 2])
        pltpu.sync_copy(acc_d, vsum_hbm.at[rg, cs, 3])
        pltpu.sync_copy(wacc_v, wsum_hbm.at[rg, cs])

    return k(feats, weights, labels)


def _tc_reduce(vsum, wsum):
    def body(vs_ref, ws_ref, vo_ref, wo_ref):
        vo_ref[...] = jnp.sum(vs_ref[...], axis=0)
        wo_ref[...] = jnp.sum(ws_ref[...], axis=(0, 1))

    return pl.pallas_call(
        body,
        out_shape=[
            jax.ShapeDtypeStruct((NCS, 4, C * 16), jnp.float32),
            jax.ShapeDtypeStruct((C * 16,), jnp.float32),
        ],
    )(vsum, wsum)


def _tc_proto(vsum, wsum, proto):
    def body(vs_ref, ws_ref, p_ref, out_ref):
        vs = jnp.concatenate(
            [vs_ref[i, j] for i in range(NCS) for j in range(4)],
            axis=1)
        ws = ws_ref[...][:, 0:1]
        vec = vs / jnp.clip(ws, EPS, None)
        old = p_ref[...]
        newp = jnp.where(ws > 0.0, MOM * old + (1.0 - MOM) * vec, old)
        nrm = jnp.sqrt(jnp.sum(newp * newp, axis=1, keepdims=True))
        out_ref[...] = newp / jnp.clip(nrm, EPS, None)

    return pl.pallas_call(
        body,
        out_shape=jax.ShapeDtypeStruct((C, D), jnp.float32),
    )(vsum, wsum, proto)


def _tc_logits(feats, p):
    BN = 4096

    def body(f_ref, p_ref, out_ref):
        f = f_ref[...]
        nrm = jnp.sqrt(jnp.sum(f * f, axis=1, keepdims=True))
        fn = f / jnp.clip(nrm, EPS, None)
        acc = lax.dot_general(
            p_ref[...], fn, (((1,), (1,)), ((), ())),
            preferred_element_type=jnp.float32,
            precision=lax.Precision.DEFAULT,
        )
        out_ref[...] = acc / TAU

    out_t = pl.pallas_call(
        body,
        grid=(N // BN,),
        in_specs=[
            pl.BlockSpec((BN, D), lambda i: (i, 0)),
            pl.BlockSpec((C, D), lambda i: (0, 0)),
        ],
        out_specs=pl.BlockSpec((C, BN), lambda i: (0, i)),
        out_shape=jax.ShapeDtypeStruct((C, N), jnp.float32),
    )(feats, p)
    return out_t.T


def kernel(feats, labels, weights, proto):
    labels = labels.astype(jnp.int32)
    vsum, wsum = _sc_scatter(feats, weights, labels)
    vsum_r, wsum_r = _tc_reduce(vsum, wsum)
    p = _tc_proto(vsum_r.reshape(NCS, 4, C, 16), wsum_r.reshape(C, 16), proto)
    return _tc_logits(feats, p)
